# Initial kernel scaffold; baseline (speedup 1.0000x reference)
import jax
import jax.numpy as jnp
from jax.experimental import pallas as pl


def kernel(token_ids, weight):
    raise NotImplementedError("write your pallas kernel here")


# speedup vs baseline: 1.0954x; 1.0954x over previous
import functools

import jax
import jax.numpy as jnp
from jax import lax
from jax.experimental import pallas as pl
from jax.experimental.pallas import tpu as pltpu
from jax.experimental.pallas import tpu_sc as plsc

_CHUNK = 1024


@functools.lru_cache(maxsize=None)
def _make_gather(B, V, D):
    info = plsc.get_sparse_core_info()
    nw = info.num_cores * info.num_subcores
    b_per_w = B // nw
    n_steps = b_per_w // _CHUNK
    mesh = plsc.VectorSubcoreMesh(core_axis_name="c", subcore_axis_name="s")

    @functools.partial(
        pl.kernel,
        mesh=mesh,
        out_type=jax.ShapeDtypeStruct((B, D), jnp.float32),
        scratch_types=[
            pltpu.VMEM((_CHUNK,), jnp.int32),
            pltpu.VMEM((_CHUNK, D), jnp.float32),
            pltpu.SemaphoreType.DMA,
        ],
        compiler_params=pltpu.CompilerParams(use_tc_tiling_on_sc=False),
    )
    def gather_kernel(idx_hbm, table_hbm, out_hbm, idx_v, rows_v, sem):
        wid = lax.axis_index("s") * info.num_cores + lax.axis_index("c")
        base = wid * b_per_w

        def body(step, carry):
            off = base + step * _CHUNK
            pltpu.sync_copy(idx_hbm.at[pl.ds(off, _CHUNK)], idx_v)
            pltpu.async_copy(table_hbm.at[idx_v], rows_v, sem).wait()
            pltpu.sync_copy(rows_v, out_hbm.at[pl.ds(off, _CHUNK)])
            return carry

        lax.fori_loop(0, n_steps, body, 0)

    return gather_kernel


def kernel(token_ids, weight):
    idx = token_ids.reshape(-1).astype(jnp.int32)
    out = _make_gather(idx.shape[0], weight.shape[0], weight.shape[1])(
        idx, weight
    )
    return out.reshape(*token_ids.shape, weight.shape[1])


# speedup vs baseline: 1.1083x; 1.0118x over previous
import functools

import jax
import jax.numpy as jnp
from jax import lax
from jax.experimental import pallas as pl
from jax.experimental.pallas import tpu as pltpu
from jax.experimental.pallas import tpu_sc as plsc

_CH = 800
_NBUF = 4


@functools.lru_cache(maxsize=None)
def _make_gather(B, V, D):
    info = plsc.get_sparse_core_info()
    nw = info.num_cores * info.num_subcores
    b_per_w = B // nw
    n_steps = b_per_w // _CH
    n_super = n_steps // _NBUF
    mesh = plsc.VectorSubcoreMesh(core_axis_name="c", subcore_axis_name="s")

    @functools.partial(
        pl.kernel,
        mesh=mesh,
        out_type=jax.ShapeDtypeStruct((B, D), jnp.float32),
        scratch_types=[
            pltpu.VMEM((b_per_w,), jnp.int32),
            *[pltpu.VMEM((_CH, D), jnp.float32) for _ in range(_NBUF)],
            *[pltpu.SemaphoreType.DMA for _ in range(2 * _NBUF)],
        ],
        compiler_params=pltpu.CompilerParams(use_tc_tiling_on_sc=False),
    )
    def gather_kernel(idx_hbm, table_hbm, out_hbm, idx_all, *rest):
        rows = rest[:_NBUF]
        sems_g = rest[_NBUF:2 * _NBUF]
        sems_o = rest[2 * _NBUF:]
        wid = lax.axis_index("s") * info.num_cores + lax.axis_index("c")
        base = wid * b_per_w
        pltpu.sync_copy(idx_hbm.at[pl.ds(base, b_per_w)], idx_all)

        def g_start(s, j):
            pltpu.async_copy(
                table_hbm.at[idx_all.at[pl.ds(s * _CH, _CH)]],
                rows[j], sems_g[j])

        def g_wait(j):
            pltpu.make_async_copy(
                out_hbm.at[pl.ds(0, _CH)], rows[j], sems_g[j]).wait()

        def o_start(s, j):
            pltpu.async_copy(
                rows[j], out_hbm.at[pl.ds(base + s * _CH, _CH)], sems_o[j])

        def o_wait(j):
            pltpu.make_async_copy(
                rows[j], out_hbm.at[pl.ds(0, _CH)], sems_o[j]).wait()

        for j in range(_NBUF):
            g_start(j, j)

        def body(ss, carry):
            for j in range(_NBUF):
                g_wait(j)
                o_start(ss * _NBUF + j, j)

            @pl.when(ss + 1 < n_super)
            def _():
                for j in range(_NBUF):
                    o_wait(j)
                    g_start((ss + 1) * _NBUF + j, j)

            return carry

        lax.fori_loop(0, n_super, body, 0)
        for j in range(_NBUF):
            o_wait(j)

    return gather_kernel


def kernel(token_ids, weight):
    idx = token_ids.reshape(-1).astype(jnp.int32)
    out = _make_gather(idx.shape[0], weight.shape[0], weight.shape[1])(
        idx, weight
    )
    return out.reshape(*token_ids.shape, weight.shape[1])
